# baseline (speedup 1.0000x reference)
import functools

import jax
import jax.numpy as jnp
from jax import lax
from jax.experimental import pallas as pl
from jax.experimental.pallas import tpu as pltpu
from jax.experimental.pallas import tpu_sc as plsc

_NUM_CORES = 2
_NUM_SUBCORES = 16
_NW = _NUM_CORES * _NUM_SUBCORES
_GCHUNK = 128

_VT = 2048
_NCHUNK = 8


def _gather_sum_sc(idx_flat, emb, B, C, D):
  per_w = B // _NW
  n_idx = per_w * C
  n_full = n_idx // _GCHUNK
  tail = n_idx - n_full * _GCHUNK

  mesh = plsc.VectorSubcoreMesh(
      core_axis_name="c", subcore_axis_name="s",
      num_cores=_NUM_CORES, num_subcores=_NUM_SUBCORES)

  @functools.partial(
      pl.kernel,
      out_type=jax.ShapeDtypeStruct((B, D), jnp.float32),
      mesh=mesh,
      compiler_params=pltpu.CompilerParams(use_tc_tiling_on_sc=False),
      scratch_types=[
          pltpu.VMEM((n_idx,), jnp.int32),
          pltpu.VMEM((n_idx, D), jnp.float32),
          pltpu.VMEM((per_w, D), jnp.float32),
          pltpu.SemaphoreType.DMA,
      ],
  )
  def gather_sum(emb_hbm, idx_hbm, out_hbm, idx_v, rows_v, acc_v, sem):
    wid = lax.axis_index("s") * _NUM_CORES + lax.axis_index("c")
    base = wid * n_idx
    pltpu.sync_copy(idx_hbm.at[pl.ds(base, n_idx)], idx_v)
    copies = []
    for j in range(n_full):
      copies.append(pltpu.async_copy(
          emb_hbm.at[idx_v.at[pl.ds(j * _GCHUNK, _GCHUNK)]],
          rows_v.at[pl.ds(j * _GCHUNK, _GCHUNK)], sem))
    if tail:
      copies.append(pltpu.async_copy(
          emb_hbm.at[idx_v.at[pl.ds(n_full * _GCHUNK, tail)]],
          rows_v.at[pl.ds(n_full * _GCHUNK, tail)], sem))
    for cp in copies:
      cp.wait()

    def row_body(r, carry):
      acc = rows_v[r * C]
      for c in range(1, C):
        acc = acc + rows_v[r * C + c]
      acc_v[r] = acc
      return carry

    lax.fori_loop(0, per_w, row_body, 0)
    pltpu.sync_copy(acc_v, out_hbm.at[pl.ds(wid * per_w, per_w)])

  return gather_sum(emb, idx_flat)


def _logits_tile(x, w, bvec):
  return lax.dot_general(
      x, w, (((1,), (1,)), ((), ())),
      preferred_element_type=jnp.float32) + bvec


def _make_fused_body(B, CB, nvt, tailw):
  nchunk = B // CB

  def fused_body(x_ref, w_ref, b_ref, o_ref, lse_out_ref,
                 obuf_ref, m_ref, s_ref, lse_ref, sem_ref):
    q = pl.program_id(0)
    j = pl.program_id(1)
    w = w_ref[pl.ds(j * _VT, _VT), :]
    bv = b_ref[pl.ds(j, 1), :]

    @pl.when(j == 0)
    def _():
      lse_ref[...] = m_ref[...] + jnp.log(s_ref[...])
      m_ref[...] = jnp.full_like(m_ref[...], -jnp.inf)
      s_ref[...] = jnp.zeros_like(s_ref[...])

    lse_out_ref[...] = jnp.broadcast_to(lse_ref[...], lse_out_ref.shape)

    @pl.when(q < nchunk)
    def _stats():
      row = jnp.minimum(q, nchunk - 1) * CB
      xs = x_ref[pl.ds(row, CB), :]
      logits = _logits_tile(xs, w, bv)
      tmax = jnp.max(logits, axis=1, keepdims=True)
      m_old = m_ref[...]
      m_new = jnp.maximum(m_old, tmax)
      s_ref[...] = (s_ref[...] * jnp.exp(m_old - m_new)
                    + jnp.sum(jnp.exp(logits - m_new), axis=1,
                              keepdims=True))
      m_ref[...] = m_new

    @pl.when(q > 0)
    def _write():
      qq = q - 1
      wk = qq * nvt + j
      slot = lax.rem(wk, 2)
      xq = x_ref[pl.ds(qq * CB, CB), :]
      logits_w = _logits_tile(xq, w, bv)

      @pl.when(jnp.logical_and(wk >= 2, j != 1))
      def _():
        pltpu.make_async_copy(
            obuf_ref.at[slot],
            o_ref.at[pl.ds(0, CB), pl.ds(0, _VT)],
            sem_ref.at[slot]).wait()

      @pl.when(jnp.logical_and(wk >= 2, j == 1))
      def _():
        pltpu.make_async_copy(
            obuf_ref.at[slot, :, pl.ds(0, tailw)],
            o_ref.at[pl.ds(0, CB), pl.ds(0, tailw)],
            sem_ref.at[slot]).wait()

      obuf_ref[slot] = logits_w - lse_ref[...]

      @pl.when(j < nvt - 1)
      def _():
        pltpu.make_async_copy(
            obuf_ref.at[slot],
            o_ref.at[pl.ds(qq * CB, CB), pl.ds(j * _VT, _VT)],
            sem_ref.at[slot]).start()

      @pl.when(j == nvt - 1)
      def _():
        pltpu.make_async_copy(
            obuf_ref.at[slot, :, pl.ds(0, tailw)],
            o_ref.at[pl.ds(qq * CB, CB), pl.ds(j * _VT, tailw)],
            sem_ref.at[slot]).start()

      @pl.when(wk == nchunk * nvt - 1)
      def _():
        pltpu.make_async_copy(
            obuf_ref.at[1 - slot],
            o_ref.at[pl.ds(0, CB), pl.ds(0, _VT)],
            sem_ref.at[1 - slot]).wait()
        pltpu.make_async_copy(
            obuf_ref.at[slot, :, pl.ds(0, tailw)],
            o_ref.at[pl.ds(0, CB), pl.ds(0, tailw)],
            sem_ref.at[slot]).wait()

  return fused_body


def _make_strip_body(stripw):
  def strip_body(main_ref, x_ref, ws_ref, bs_ref, lse_ref, o_ref):
    del main_ref
    logits = _logits_tile(x_ref[...], ws_ref[...], bs_ref[...])
    o_ref[...] = logits - lse_ref[...][:, 0:1]

  return strip_body


def kernel(inputs, emb, W, b):
  B, C = inputs.shape
  V, D = emb.shape
  nvt = pl.cdiv(V, _VT)
  VP = nvt * _VT
  CB = B // _NCHUNK
  aligned = (V // 128) * 128
  stripw = V - aligned
  tailw = aligned - (nvt - 1) * _VT

  idx_flat = inputs.reshape(B * C).astype(jnp.int32)
  x = _gather_sum_sc(idx_flat, emb, B, C, D)

  W_pad = jnp.pad(W, ((0, VP - V), (0, 0)))
  b_pad = jnp.pad(b, (0, VP - V), constant_values=-1e30).reshape(nvt, _VT)

  main, lse_all = pl.pallas_call(
      _make_fused_body(B, CB, nvt, tailw),
      grid=(_NCHUNK + 1, nvt),
      in_specs=[
          pl.BlockSpec((B, D), lambda q, j: (0, 0)),
          pl.BlockSpec((VP, D), lambda q, j: (0, 0)),
          pl.BlockSpec((nvt, _VT), lambda q, j: (0, 0)),
      ],
      out_specs=[
          pl.BlockSpec(memory_space=pl.ANY),
          pl.BlockSpec((CB, 128), lambda q, j: (jnp.maximum(q - 1, 0), 0)),
      ],
      out_shape=[
          jax.ShapeDtypeStruct((B, V), jnp.float32),
          jax.ShapeDtypeStruct((B, 128), jnp.float32),
      ],
      scratch_shapes=[
          pltpu.VMEM((2, CB, _VT), jnp.float32),
          pltpu.VMEM((CB, 1), jnp.float32),
          pltpu.VMEM((CB, 1), jnp.float32),
          pltpu.VMEM((CB, 1), jnp.float32),
          pltpu.SemaphoreType.DMA((2,)),
      ],
  )(x, W_pad, b_pad)

  Ws = jnp.pad(lax.slice(W, (aligned, 0), (V, D)), ((0, 128 - stripw), (0, 0)))
  bs = jnp.pad(lax.slice(b, (aligned,), (V,)),
               (0, 128 - stripw)).reshape(1, 128)
  log_probs = pl.pallas_call(
      _make_strip_body(stripw),
      grid=(1,),
      in_specs=[
          pl.BlockSpec(memory_space=pl.ANY),
          pl.BlockSpec((B, D), lambda i: (0, 0)),
          pl.BlockSpec((128, D), lambda i: (0, 0)),
          pl.BlockSpec((1, 128), lambda i: (0, 0)),
          pl.BlockSpec((B, 128), lambda i: (0, 0)),
      ],
      out_specs=pl.BlockSpec((B, 128), lambda i: (0, aligned // 128)),
      out_shape=jax.ShapeDtypeStruct((B, V), jnp.float32),
      input_output_aliases={0: 0},
  )(main, x, Ws, bs, lse_all)

  return log_probs


# speedup vs baseline: 1.3724x; 1.3724x over previous
import functools

import jax
import jax.numpy as jnp
from jax import lax
from jax.experimental import pallas as pl
from jax.experimental.pallas import tpu as pltpu
from jax.experimental.pallas import tpu_sc as plsc

_NUM_CORES = 2
_NUM_SUBCORES = 16
_NW = _NUM_CORES * _NUM_SUBCORES
_GCHUNK = 128

_VTS = 2048
_VTW = 3584


def _gather_sum_sc(idx_flat, emb, B, C, D):
  per_w = B // _NW
  n_idx = per_w * C
  n_full = n_idx // _GCHUNK
  tail = n_idx - n_full * _GCHUNK

  mesh = plsc.VectorSubcoreMesh(
      core_axis_name="c", subcore_axis_name="s",
      num_cores=_NUM_CORES, num_subcores=_NUM_SUBCORES)

  @functools.partial(
      pl.kernel,
      out_type=jax.ShapeDtypeStruct((B, D), jnp.float32),
      mesh=mesh,
      compiler_params=pltpu.CompilerParams(use_tc_tiling_on_sc=False),
      scratch_types=[
          pltpu.VMEM((n_idx,), jnp.int32),
          pltpu.VMEM((n_idx, D), jnp.float32),
          pltpu.VMEM((per_w, D), jnp.float32),
          pltpu.SemaphoreType.DMA,
      ],
  )
  def gather_sum(emb_hbm, idx_hbm, out_hbm, idx_v, rows_v, acc_v, sem):
    wid = lax.axis_index("s") * _NUM_CORES + lax.axis_index("c")
    base = wid * n_idx
    pltpu.sync_copy(idx_hbm.at[pl.ds(base, n_idx)], idx_v)
    copies = []
    for j in range(n_full):
      copies.append(pltpu.async_copy(
          emb_hbm.at[idx_v.at[pl.ds(j * _GCHUNK, _GCHUNK)]],
          rows_v.at[pl.ds(j * _GCHUNK, _GCHUNK)], sem))
    if tail:
      copies.append(pltpu.async_copy(
          emb_hbm.at[idx_v.at[pl.ds(n_full * _GCHUNK, tail)]],
          rows_v.at[pl.ds(n_full * _GCHUNK, tail)], sem))
    for cp in copies:
      cp.wait()

    def row_body(r, carry):
      acc = rows_v[r * C]
      for c in range(1, C):
        acc = acc + rows_v[r * C + c]
      acc_v[r] = acc
      return carry

    lax.fori_loop(0, per_w, row_body, 0)
    pltpu.sync_copy(acc_v, out_hbm.at[pl.ds(wid * per_w, per_w)])

  return gather_sum(emb, idx_flat)


def _logits_tile(x, w, bvec):
  return lax.dot_general(
      x, w, (((1,), (1,)), ((), ())),
      preferred_element_type=jnp.float32) + bvec


def _stats_body(x_ref, w_ref, b_ref, lse_ref, m_ref, s_ref):
  j = pl.program_id(0)
  nj = pl.num_programs(0)
  logits = _logits_tile(x_ref[...], w_ref[...], b_ref[...])
  tmax = jnp.max(logits, axis=1, keepdims=True)

  @pl.when(j == 0)
  def _():
    m_ref[...] = jnp.full_like(m_ref[...], -jnp.inf)
    s_ref[...] = jnp.zeros_like(s_ref[...])

  m_old = m_ref[...]
  m_new = jnp.maximum(m_old, tmax)
  s_ref[...] = (s_ref[...] * jnp.exp(m_old - m_new)
                + jnp.sum(jnp.exp(logits - m_new), axis=1, keepdims=True))
  m_ref[...] = m_new

  @pl.when(j == nj - 1)
  def _():
    lse_ref[...] = jnp.broadcast_to(
        m_ref[...] + jnp.log(s_ref[...]), lse_ref.shape)


def _out_body(x_ref, w_ref, b_ref, lse_ref, o_ref):
  o_ref[...] = (_logits_tile(x_ref[...], w_ref[...], b_ref[...])
                - lse_ref[...][:, 0:1])


def kernel(inputs, emb, W, b):
  B, C = inputs.shape
  V, D = emb.shape
  nvs = pl.cdiv(V, _VTS)
  VP = nvs * _VTS
  nvw = VP // _VTW

  idx_flat = inputs.reshape(B * C).astype(jnp.int32)
  x = _gather_sum_sc(idx_flat, emb, B, C, D)

  W_pad = jnp.pad(W, ((0, VP - V), (0, 0)))
  b_pad = jnp.pad(b, (0, VP - V), constant_values=-1e30).reshape(1, VP)

  lse = pl.pallas_call(
      _stats_body,
      grid=(nvs,),
      in_specs=[
          pl.BlockSpec((B, D), lambda j: (0, 0)),
          pl.BlockSpec((_VTS, D), lambda j: (j, 0)),
          pl.BlockSpec((1, _VTS), lambda j: (0, j)),
      ],
      out_specs=pl.BlockSpec((B, 128), lambda j: (0, 0)),
      out_shape=jax.ShapeDtypeStruct((B, 128), jnp.float32),
      scratch_shapes=[
          pltpu.VMEM((B, 1), jnp.float32),
          pltpu.VMEM((B, 1), jnp.float32),
      ],
  )(x, W_pad, b_pad)

  log_probs = pl.pallas_call(
      _out_body,
      grid=(nvw,),
      in_specs=[
          pl.BlockSpec((B, D), lambda j: (0, 0)),
          pl.BlockSpec((_VTW, D), lambda j: (j, 0)),
          pl.BlockSpec((1, _VTW), lambda j: (0, j)),
          pl.BlockSpec((B, 128), lambda j: (0, 0)),
      ],
      out_specs=pl.BlockSpec((B, _VTW), lambda j: (0, j)),
      out_shape=jax.ShapeDtypeStruct((B, V), jnp.float32),
      compiler_params=pltpu.CompilerParams(
          dimension_semantics=("parallel",)),
  )(x, W_pad, b_pad, lse)

  return log_probs


# speedup vs baseline: 1.3724x; 1.0000x over previous
import functools

import jax
import jax.numpy as jnp
from jax import lax
from jax.experimental import pallas as pl
from jax.experimental.pallas import tpu as pltpu
from jax.experimental.pallas import tpu_sc as plsc

_NUM_CORES = 2
_NUM_SUBCORES = 16
_NW = _NUM_CORES * _NUM_SUBCORES
_GCHUNK = 128

_VTS = 2048
_VTW = 3584


def _gather_sum_sc(idx_flat, emb, B, C, D):
  per_w = B // _NW
  n_idx = per_w * C
  n_full = n_idx // _GCHUNK
  tail = n_idx - n_full * _GCHUNK

  mesh = plsc.VectorSubcoreMesh(
      core_axis_name="c", subcore_axis_name="s",
      num_cores=_NUM_CORES, num_subcores=_NUM_SUBCORES)

  @functools.partial(
      pl.kernel,
      out_type=jax.ShapeDtypeStruct((B, D), jnp.float32),
      mesh=mesh,
      compiler_params=pltpu.CompilerParams(use_tc_tiling_on_sc=False),
      scratch_types=[
          pltpu.VMEM((n_idx,), jnp.int32),
          pltpu.VMEM((n_idx, D), jnp.float32),
          pltpu.VMEM((per_w, D), jnp.float32),
          pltpu.SemaphoreType.DMA,
      ],
  )
  def gather_sum(emb_hbm, idx_hbm, out_hbm, idx_v, rows_v, acc_v, sem):
    wid = lax.axis_index("s") * _NUM_CORES + lax.axis_index("c")
    base = wid * n_idx
    pltpu.sync_copy(idx_hbm.at[pl.ds(base, n_idx)], idx_v)
    copies = []
    for j in range(n_full):
      copies.append(pltpu.async_copy(
          emb_hbm.at[idx_v.at[pl.ds(j * _GCHUNK, _GCHUNK)]],
          rows_v.at[pl.ds(j * _GCHUNK, _GCHUNK)], sem))
    if tail:
      copies.append(pltpu.async_copy(
          emb_hbm.at[idx_v.at[pl.ds(n_full * _GCHUNK, tail)]],
          rows_v.at[pl.ds(n_full * _GCHUNK, tail)], sem))
    for cp in copies:
      cp.wait()

    def row_body(r, carry):
      acc = rows_v[r * C]
      for c in range(1, C):
        acc = acc + rows_v[r * C + c]
      acc_v[r] = acc
      return carry

    lax.fori_loop(0, per_w, row_body, 0)
    pltpu.sync_copy(acc_v, out_hbm.at[pl.ds(wid * per_w, per_w)])

  return gather_sum(emb, idx_flat)


def _logits_tile(x, w, bvec):
  return lax.dot_general(
      x, w, (((1,), (1,)), ((), ())),
      preferred_element_type=jnp.float32) + bvec


def _stats_body(x_ref, w_ref, b_ref, bd_ref, lse_ref, s_ref):
  j = pl.program_id(0)
  nj = pl.num_programs(0)
  logits = _logits_tile(x_ref[...], w_ref[...], b_ref[...])
  bd = bd_ref[...][:, 0:1]

  @pl.when(j == 0)
  def _():
    s_ref[...] = jnp.zeros_like(s_ref[...])

  s_ref[...] = (s_ref[...]
                + jnp.sum(jnp.exp(logits - bd), axis=1, keepdims=True))

  @pl.when(j == nj - 1)
  def _():
    lse_ref[...] = jnp.broadcast_to(
        bd + jnp.log(s_ref[...]), lse_ref.shape)


def _out_body(x_ref, w_ref, b_ref, lse_ref, o_ref):
  o_ref[...] = (_logits_tile(x_ref[...], w_ref[...], b_ref[...])
                - lse_ref[...][:, 0:1])


def kernel(inputs, emb, W, b):
  B, C = inputs.shape
  V, D = emb.shape
  nvs = pl.cdiv(V, _VTS)
  VP = nvs * _VTS
  nvw = VP // _VTW

  idx_flat = inputs.reshape(B * C).astype(jnp.int32)
  x = _gather_sum_sc(idx_flat, emb, B, C, D)

  W_pad = jnp.pad(W, ((0, VP - V), (0, 0)))
  b_pad = jnp.pad(b, (0, VP - V), constant_values=-1e30).reshape(1, VP)

  xnorm = jnp.sqrt(jnp.sum(x * x, axis=1, keepdims=True))
  wmax = jnp.sqrt(jnp.max(jnp.sum(W * W, axis=1)))
  bound = jnp.broadcast_to(xnorm * wmax + jnp.max(b), (B, 128))

  lse = pl.pallas_call(
      _stats_body,
      grid=(nvs,),
      in_specs=[
          pl.BlockSpec((B, D), lambda j: (0, 0)),
          pl.BlockSpec((_VTS, D), lambda j: (j, 0)),
          pl.BlockSpec((1, _VTS), lambda j: (0, j)),
          pl.BlockSpec((B, 128), lambda j: (0, 0)),
      ],
      out_specs=pl.BlockSpec((B, 128), lambda j: (0, 0)),
      out_shape=jax.ShapeDtypeStruct((B, 128), jnp.float32),
      scratch_shapes=[
          pltpu.VMEM((B, 1), jnp.float32),
      ],
  )(x, W_pad, b_pad, bound)

  log_probs = pl.pallas_call(
      _out_body,
      grid=(nvw,),
      in_specs=[
          pl.BlockSpec((B, D), lambda j: (0, 0)),
          pl.BlockSpec((_VTW, D), lambda j: (j, 0)),
          pl.BlockSpec((1, _VTW), lambda j: (0, j)),
          pl.BlockSpec((B, 128), lambda j: (0, 0)),
      ],
      out_specs=pl.BlockSpec((B, _VTW), lambda j: (0, j)),
      out_shape=jax.ShapeDtypeStruct((B, V), jnp.float32),
      compiler_params=pltpu.CompilerParams(
          dimension_semantics=("parallel",)),
  )(x, W_pad, b_pad, lse)

  return log_probs
